# baseline (speedup 1.0000x reference)
import functools

import jax
import jax.numpy as jnp
from jax.experimental import pallas as pl
from jax.experimental.pallas import tpu as pltpu

_SKIP = 10
_LABEL_THRESHOLD = 100.0
_INLIER_THRESHOLD = 0.9
_PER_THRESHOLD = 0.01

_DNT = (((1,), (1,)), ((), ()))
_DNN = (((1,), (0,)), ((), ()))


def _hough_body(nx, ny, n, nb, cp, w, va_ref, vb_ref, labr_ref, ext_ref,
                meta_ref, out_ref, votes_scr, zs_scr, cnt_scr):
    f32 = jnp.float32
    i = pl.program_id(0)
    nprep = ny // 2

    @pl.when(i < nprep)
    def _prep_and_accumulate():
        w_i = jax.lax.broadcasted_iota(jnp.int32, (w, nx), 0)
        j_i = jax.lax.broadcasted_iota(jnp.int32, (w, nx), 1)
        sel = (w_i == j_i * _SKIP).astype(f32)

        lab_chunk = labr_ref[0]
        sub_i = jax.lax.broadcasted_iota(jnp.int32, (cp, 2 * nx), 0)

        k_i = jax.lax.broadcasted_iota(jnp.int32, (n, 1), 0)
        xk = ((k_i % nx) * _SKIP).astype(f32)
        yk = ((k_i // nx) * _SKIP).astype(f32)
        p_i = jax.lax.broadcasted_iota(jnp.int32, (1, 2 * nx), 1) + i * 2 * nx
        xp = ((p_i % nx) * _SKIP).astype(f32)
        yp = ((p_i // nx) * _SKIP).astype(f32)
        dx = xk - xp
        dy = yk - yp
        thr9 = f32(_INLIER_THRESHOLD) * (jnp.sqrt(dx * dx + dy * dy)
                                         + f32(1e-6))

        for b in range(nb):
            subs = []
            for j in range(3):
                pa = jax.lax.dot_general(va_ref[b, :, j, 0, 0, :], sel, _DNN,
                                         preferred_element_type=f32)
                pb = jax.lax.dot_general(vb_ref[b, :, j, 0, 0, :], sel, _DNN,
                                         preferred_element_type=f32)
                subs.append(jnp.concatenate([pa, pb], axis=1))
            u_c, v_c, z_c = subs

            oh = (lab_chunk[b:b + 1, :] == sub_i).astype(f32)
            zoh = oh * z_c
            u_row = jnp.sum(jnp.where(oh > 0, u_c, 0.0), axis=0,
                            keepdims=True)
            v_row = jnp.sum(jnp.where(oh > 0, v_c, 0.0), axis=0,
                            keepdims=True)
            nrm = jnp.sqrt(u_row * u_row + v_row * v_row) + f32(1e-6)
            un = u_row / nrm
            vn = v_row / nrm

            ind = (dx * un + dy * vn > thr9).astype(f32)
            dv = jax.lax.dot_general(ind, oh, _DNT,
                                     preferred_element_type=f32)
            dz = jax.lax.dot_general(ind, zoh, _DNT,
                                     preferred_element_type=f32)
            dc = jax.lax.dot_general(jnp.ones((1, 2 * nx), f32), oh, _DNT,
                                     preferred_element_type=f32)

            votes_scr[b] = jnp.where(i == 0, dv, votes_scr[b] + dv)
            zs_scr[b] = jnp.where(i == 0, dz, zs_scr[b] + dz)
            cnt_scr[b] = jnp.where(i == 0, dc, cnt_scr[b] + dc)

    @pl.when(i == nprep)
    def _reduce_and_assemble():
        k_idx = jax.lax.broadcasted_iota(jnp.int32, (n, cp), 0)
        cnum = jax.lax.broadcasted_iota(jnp.int32, (1, cp), 1).astype(f32)
        zero = jnp.zeros((1, cp), f32)
        e0 = ext_ref[0:1, :]
        e1 = ext_ref[1:2, :]
        e2 = ext_ref[2:3, :]
        diam = jnp.sqrt(e0 * e0 + e1 * e1 + e2 * e2)
        for b in range(nb):
            votes = votes_scr[b]
            zs = zs_scr[b]
            count = cnt_scr[b]
            vmax = jnp.max(votes, axis=0, keepdims=True)
            best = jnp.min(jnp.where(votes == vmax, k_idx, n), axis=0,
                           keepdims=True)
            zbest = jnp.sum(jnp.where(k_idx == best, zs, 0.0), axis=0,
                            keepdims=True)

            bx = ((best % nx) * _SKIP).astype(f32)
            by = ((best // nx) * _SKIP).astype(f32)
            denom = vmax + f32(1e-6)
            tz = jnp.abs(zbest / denom) + f32(0.5)
            fxv = meta_ref[b, 0:1, :]
            pxv = meta_ref[b, 1:2, :]
            fyv = meta_ref[b, 2:3, :]
            pyv = meta_ref[b, 3:4, :]
            bw = fxv * diam / tz
            bh = fyv * diam / tz
            thr = count * f32(_PER_THRESHOLD)
            valid = jnp.logical_and(count > f32(_LABEL_THRESHOLD),
                                    vmax >= thr).astype(f32)
            tx = (bx - pxv) * tz / fxv
            ty = (by - pyv) * tz / fyv
            out_ref[b] = jnp.concatenate([
                jnp.full((1, cp), float(b), f32) * valid,
                cnum * valid,
                (bx - bw * 0.5) * valid,
                (by - bh * 0.5) * valid,
                (bx + bw * 0.5) * valid,
                (by + bh * 0.5) * valid,
                vmax * valid,
                valid,
                zero, zero, zero,
                tx * valid,
                ty * valid,
                tz * valid,
                zero, zero,
            ], axis=0)


def kernel(label, vertex, meta_data, extents):
    b, h, w = label.shape
    c = extents.shape[0]
    ny = -(-h // _SKIP)
    nx = -(-w // _SKIP)
    n = ny * nx
    f32 = jnp.float32

    nstep = (ny // 2) + 1
    lab_steps = (label[:, ::_SKIP, ::_SKIP].reshape(b, ny // 2, 2 * nx)
                 .transpose(1, 0, 2).astype(jnp.int32))
    lab_steps = jnp.concatenate(
        [lab_steps, lab_steps[-1:]], axis=0)
    v5 = vertex.reshape(b, c, 3, h, 1, w)
    ext3 = extents.T.astype(f32)
    meta_bc = jnp.broadcast_to(
        meta_data[:, jnp.array([0, 2, 4, 5])][:, :, None], (b, 4, c))

    nprep = ny // 2
    row_a = lambda i: (0, 0, 0, jnp.minimum(i * 2, ny - 1) * _SKIP, 0, 0)
    row_b = lambda i: (0, 0, 0, jnp.minimum(i * 2 + 1, ny - 1) * _SKIP, 0, 0)

    out = pl.pallas_call(
        functools.partial(_hough_body, nx, ny, n, b, c, w),
        grid=(nprep + 1,),
        in_specs=[
            pl.BlockSpec((b, c, 3, 1, 1, w), row_a),
            pl.BlockSpec((b, c, 3, 1, 1, w), row_b),
            pl.BlockSpec((1, b, 2 * nx), lambda i: (i, 0, 0)),
            pl.BlockSpec((3, c), lambda i: (0, 0)),
            pl.BlockSpec((b, 4, c), lambda i: (0, 0, 0)),
        ],
        out_specs=pl.BlockSpec((b, 16, c), lambda i: (0, 0, 0)),
        out_shape=jax.ShapeDtypeStruct((b, 16, c), f32),
        scratch_shapes=[
            pltpu.VMEM((b, n, c), f32),
            pltpu.VMEM((b, n, c), f32),
            pltpu.VMEM((b, 1, c), f32),
        ],
    )(v5, v5, lab_steps, ext3, meta_bc)

    top_box = out[:, 0:7, :].transpose(0, 2, 1).reshape(b * c, 7)
    top_pose = out[:, 7:14, :].transpose(0, 2, 1).reshape(b * c, 7)
    return top_box, top_pose


# speedup vs baseline: 4.8714x; 4.8714x over previous
import functools

import jax
import jax.numpy as jnp
from jax.experimental import pallas as pl
from jax.experimental.pallas import tpu as pltpu

_SKIP = 10
_LABEL_THRESHOLD = 100.0
_INLIER_THRESHOLD = 0.9
_PER_THRESHOLD = 0.01
_KT = 256

_DNT = (((1,), (1,)), ((), ()))


def _hough_body(nx, n, nb, cp, uvz_ref, labr_ref, ext_ref,
                meta_ref, out_ref, votes_scr, zs_scr):
    f32 = jnp.float32

    sub_i = jax.lax.broadcasted_iota(jnp.int32, (cp, n), 0)
    ones_row = jnp.ones((1, n), f32)
    ohs, zohs, uns, vns, counts = [], [], [], [], []
    for b in range(nb):
        oh_t = (labr_ref[b] == sub_i).astype(f32)
        ohs.append(oh_t)
        zohs.append(oh_t * uvz_ref[b, :, 2, :])
        counts.append(jax.lax.dot_general(
            ones_row, oh_t, _DNT, preferred_element_type=f32))
        u_row = jnp.sum(jnp.where(oh_t > 0, uvz_ref[b, :, 0, :], 0.0), axis=0,
                        keepdims=True)
        v_row = jnp.sum(jnp.where(oh_t > 0, uvz_ref[b, :, 1, :], 0.0), axis=0,
                        keepdims=True)
        nrm = jnp.sqrt(u_row * u_row + v_row * v_row) + f32(1e-6)
        uns.append(u_row / nrm)
        vns.append(v_row / nrm)

    p_i = jax.lax.broadcasted_iota(jnp.int32, (1, n), 1)
    xp = ((p_i % nx) * _SKIP).astype(f32)
    yp = ((p_i // nx) * _SKIP).astype(f32)

    def tile(t, carry):
        k_i = jax.lax.broadcasted_iota(jnp.int32, (_KT, 1), 0) + t * _KT
        xk = ((k_i % nx) * _SKIP).astype(f32)
        yk = ((k_i // nx) * _SKIP).astype(f32)
        dx = xk - xp
        dy = yk - yp
        thr9 = f32(_INLIER_THRESHOLD) * (jnp.sqrt(dx * dx + dy * dy)
                                         + f32(1e-6))
        for b in range(nb):
            ind = (dx * uns[b] + dy * vns[b] > thr9).astype(f32)
            votes_scr[b, pl.ds(t * _KT, _KT), :] = jax.lax.dot_general(
                ind, ohs[b], _DNT, preferred_element_type=f32)
            zs_scr[b, pl.ds(t * _KT, _KT), :] = jax.lax.dot_general(
                ind, zohs[b], _DNT, preferred_element_type=f32)
        return carry

    jax.lax.fori_loop(0, n // _KT, tile, 0)

    k_idx = jax.lax.broadcasted_iota(jnp.int32, (n, cp), 0)
    cnum = jax.lax.broadcasted_iota(jnp.int32, (1, cp), 1).astype(f32)
    zero = jnp.zeros((1, cp), f32)
    e0 = ext_ref[0:1, :]
    e1 = ext_ref[1:2, :]
    e2 = ext_ref[2:3, :]
    diam = jnp.sqrt(e0 * e0 + e1 * e1 + e2 * e2)
    for b in range(nb):
        votes = votes_scr[b]
        zs = zs_scr[b]
        vmax = jnp.max(votes, axis=0, keepdims=True)
        best = jnp.min(jnp.where(votes == vmax, k_idx, n), axis=0,
                       keepdims=True)
        zbest = jnp.sum(jnp.where(k_idx == best, zs, 0.0), axis=0,
                        keepdims=True)
        count = counts[b]

        bx = ((best % nx) * _SKIP).astype(f32)
        by = ((best // nx) * _SKIP).astype(f32)
        denom = vmax + f32(1e-6)
        tz = jnp.abs(zbest / denom) + f32(0.5)
        fxv = meta_ref[b, 0:1, :]
        pxv = meta_ref[b, 1:2, :]
        fyv = meta_ref[b, 2:3, :]
        pyv = meta_ref[b, 3:4, :]
        bw = fxv * diam / tz
        bh = fyv * diam / tz
        thr = count * f32(_PER_THRESHOLD)
        valid = jnp.logical_and(count > f32(_LABEL_THRESHOLD),
                                vmax >= thr).astype(f32)
        tx = (bx - pxv) * tz / fxv
        ty = (by - pyv) * tz / fyv
        out_ref[b] = jnp.concatenate([
            jnp.full((1, cp), float(b), f32) * valid,
            cnum * valid,
            (bx - bw * 0.5) * valid,
            (by - bh * 0.5) * valid,
            (bx + bw * 0.5) * valid,
            (by + bh * 0.5) * valid,
            vmax * valid,
            valid,
            zero, zero, zero,
            tx * valid,
            ty * valid,
            tz * valid,
            zero, zero,
        ], axis=0)


def kernel(label, vertex, meta_data, extents):
    b, h, w = label.shape
    c = extents.shape[0]
    ny = -(-h // _SKIP)
    nx = -(-w // _SKIP)
    n = ny * nx
    f32 = jnp.float32

    lab_r = label[:, ::_SKIP, ::_SKIP].reshape(b, 1, n).astype(jnp.int32)
    vs = vertex[:, :, ::_SKIP, ::_SKIP].reshape(b, c, 3, n)
    ext3 = extents.T.astype(f32)
    meta_bc = jnp.broadcast_to(
        meta_data[:, jnp.array([0, 2, 4, 5])][:, :, None], (b, 4, c))

    out = pl.pallas_call(
        functools.partial(_hough_body, nx, n, b, c),
        in_specs=[
            pl.BlockSpec((b, c, 3, n), lambda: (0, 0, 0, 0)),
            pl.BlockSpec((b, 1, n), lambda: (0, 0, 0)),
            pl.BlockSpec((3, c), lambda: (0, 0)),
            pl.BlockSpec((b, 4, c), lambda: (0, 0, 0)),
        ],
        out_specs=pl.BlockSpec((b, 16, c), lambda: (0, 0, 0)),
        out_shape=jax.ShapeDtypeStruct((b, 16, c), f32),
        scratch_shapes=[
            pltpu.VMEM((b, n, c), f32),
            pltpu.VMEM((b, n, c), f32),
        ],
    )(vs, lab_r, ext3, meta_bc)

    top_box = out[:, 0:7, :].transpose(0, 2, 1).reshape(b * c, 7)
    top_pose = out[:, 7:14, :].transpose(0, 2, 1).reshape(b * c, 7)
    return top_box, top_pose


# speedup vs baseline: 4.8977x; 1.0054x over previous
import functools

import jax
import jax.numpy as jnp
from jax.experimental import pallas as pl
from jax.experimental.pallas import tpu as pltpu

_SKIP = 10
_LABEL_THRESHOLD = 100.0
_INLIER_THRESHOLD = 0.9
_PER_THRESHOLD = 0.01
_KT = 512

_DNT = (((1,), (1,)), ((), ()))


def _hough_body(nx, n, nb, cp, uvz_ref, labr_ref, ext_ref,
                meta_ref, out_ref, votes_scr, zs_scr):
    f32 = jnp.float32

    sub_i = jax.lax.broadcasted_iota(jnp.int32, (cp, n), 0)
    ones_row = jnp.ones((1, n), f32)
    ohs, zohs, uns, vns, counts = [], [], [], [], []
    for b in range(nb):
        oh_t = (labr_ref[b] == sub_i).astype(f32)
        ohs.append(oh_t)
        zohs.append(oh_t * uvz_ref[b, :, 2, :])
        counts.append(jax.lax.dot_general(
            ones_row, oh_t, _DNT, preferred_element_type=f32))
        u_row = jnp.sum(jnp.where(oh_t > 0, uvz_ref[b, :, 0, :], 0.0), axis=0,
                        keepdims=True)
        v_row = jnp.sum(jnp.where(oh_t > 0, uvz_ref[b, :, 1, :], 0.0), axis=0,
                        keepdims=True)
        nrm = jnp.sqrt(u_row * u_row + v_row * v_row) + f32(1e-6)
        uns.append(u_row / nrm)
        vns.append(v_row / nrm)

    p_i = jax.lax.broadcasted_iota(jnp.int32, (1, n), 1)
    xp = ((p_i % nx) * _SKIP).astype(f32)
    yp = ((p_i // nx) * _SKIP).astype(f32)

    def tile(t, carry):
        k_i = jax.lax.broadcasted_iota(jnp.int32, (_KT, 1), 0) + t * _KT
        xk = ((k_i % nx) * _SKIP).astype(f32)
        yk = ((k_i // nx) * _SKIP).astype(f32)
        dx = xk - xp
        dy = yk - yp
        thr9 = f32(_INLIER_THRESHOLD) * (jnp.sqrt(dx * dx + dy * dy)
                                         + f32(1e-6))
        for b in range(nb):
            ind = (dx * uns[b] + dy * vns[b] > thr9).astype(f32)
            votes_scr[b, pl.ds(t * _KT, _KT), :] = jax.lax.dot_general(
                ind, ohs[b], _DNT, preferred_element_type=f32)
            zs_scr[b, pl.ds(t * _KT, _KT), :] = jax.lax.dot_general(
                ind, zohs[b], _DNT, preferred_element_type=f32)
        return carry

    jax.lax.fori_loop(0, n // _KT, tile, 0)

    k_idx = jax.lax.broadcasted_iota(jnp.int32, (n, cp), 0)
    cnum = jax.lax.broadcasted_iota(jnp.int32, (1, cp), 1).astype(f32)
    zero = jnp.zeros((1, cp), f32)
    e0 = ext_ref[0:1, :]
    e1 = ext_ref[1:2, :]
    e2 = ext_ref[2:3, :]
    diam = jnp.sqrt(e0 * e0 + e1 * e1 + e2 * e2)
    for b in range(nb):
        votes = votes_scr[b]
        zs = zs_scr[b]
        vmax = jnp.max(votes, axis=0, keepdims=True)
        best = jnp.min(jnp.where(votes == vmax, k_idx, n), axis=0,
                       keepdims=True)
        zbest = jnp.sum(jnp.where(k_idx == best, zs, 0.0), axis=0,
                        keepdims=True)
        count = counts[b]

        bx = ((best % nx) * _SKIP).astype(f32)
        by = ((best // nx) * _SKIP).astype(f32)
        denom = vmax + f32(1e-6)
        tz = jnp.abs(zbest / denom) + f32(0.5)
        fxv = meta_ref[b, 0:1, :]
        pxv = meta_ref[b, 1:2, :]
        fyv = meta_ref[b, 2:3, :]
        pyv = meta_ref[b, 3:4, :]
        bw = fxv * diam / tz
        bh = fyv * diam / tz
        thr = count * f32(_PER_THRESHOLD)
        valid = jnp.logical_and(count > f32(_LABEL_THRESHOLD),
                                vmax >= thr).astype(f32)
        tx = (bx - pxv) * tz / fxv
        ty = (by - pyv) * tz / fyv
        out_ref[b] = jnp.concatenate([
            jnp.full((1, cp), float(b), f32) * valid,
            cnum * valid,
            (bx - bw * 0.5) * valid,
            (by - bh * 0.5) * valid,
            (bx + bw * 0.5) * valid,
            (by + bh * 0.5) * valid,
            vmax * valid,
            valid,
            zero, zero, zero,
            tx * valid,
            ty * valid,
            tz * valid,
            zero, zero,
        ], axis=0)


def kernel(label, vertex, meta_data, extents):
    b, h, w = label.shape
    c = extents.shape[0]
    ny = -(-h // _SKIP)
    nx = -(-w // _SKIP)
    n = ny * nx
    f32 = jnp.float32

    lab_r = label[:, ::_SKIP, ::_SKIP].reshape(b, 1, n).astype(jnp.int32)
    vs = vertex[:, :, ::_SKIP, ::_SKIP].reshape(b, c, 3, n)
    ext3 = extents.T.astype(f32)
    meta_bc = jnp.broadcast_to(
        meta_data[:, jnp.array([0, 2, 4, 5])][:, :, None], (b, 4, c))

    out = pl.pallas_call(
        functools.partial(_hough_body, nx, n, b, c),
        in_specs=[
            pl.BlockSpec((b, c, 3, n), lambda: (0, 0, 0, 0)),
            pl.BlockSpec((b, 1, n), lambda: (0, 0, 0)),
            pl.BlockSpec((3, c), lambda: (0, 0)),
            pl.BlockSpec((b, 4, c), lambda: (0, 0, 0)),
        ],
        out_specs=pl.BlockSpec((b, 16, c), lambda: (0, 0, 0)),
        out_shape=jax.ShapeDtypeStruct((b, 16, c), f32),
        scratch_shapes=[
            pltpu.VMEM((b, n, c), f32),
            pltpu.VMEM((b, n, c), f32),
        ],
    )(vs, lab_r, ext3, meta_bc)

    top_box = out[:, 0:7, :].transpose(0, 2, 1).reshape(b * c, 7)
    top_pose = out[:, 7:14, :].transpose(0, 2, 1).reshape(b * c, 7)
    return top_box, top_pose
